# baseline (speedup 1.0000x reference)
import functools

import jax
import jax.numpy as jnp
from jax import lax
from jax.experimental import pallas as pl
from jax.experimental.pallas import tpu as pltpu
from jax.experimental.pallas import tpu_sc as plsc

RANK = 16
GRID = 512
N_PTS = 262144
HW = GRID * GRID
NUM_CORES = 2
NUM_SUBCORES = 16
NUM_WORKERS = NUM_CORES * NUM_SUBCORES
PPW = N_PTS // NUM_WORKERS
CHUNK = 128
NCHUNK = PPW // CHUNK
LANES = 16
TCELLS = 2048
TPITCH = TCELLS + 8
CPW = HW // NUM_WORKERS


def _tbody(pXY, pXZ, pYZ, lns48, tbl, tinA, tinB, tout, semA, semB):
    cid = lax.axis_index("c")
    sid = lax.axis_index("s")
    wid = sid * NUM_CORES + cid
    planes = (pXY, pXZ, pYZ)
    bufs = ((tinA, semA), (tinB, semB))
    iotaT = lax.iota(jnp.int32, LANES) * TPITCH

    def fire(k, buf, sem):
        t, s = divmod(k, CPW // TCELLS)
        pref = planes[t]
        cell0 = wid * CPW + s * TCELLS
        for r in range(RANK):
            pltpu.async_copy(pref.at[r, pl.ds(cell0, TCELLS)],
                             buf.at[pl.ds(r * TPITCH, TCELLS)], sem)

    def drain(k, buf, sem):
        t, s = divmod(k, CPW // TCELLS)
        pref = planes[t]
        cell0 = wid * CPW + s * TCELLS
        for r in range(RANK):
            pltpu.make_async_copy(pref.at[r, pl.ds(cell0, TCELLS)],
                                  buf.at[pl.ds(r * TPITCH, TCELLS)],
                                  sem).wait()

    def trans_out(k, buf):
        @plsc.parallel_loop(0, TCELLS, unroll=8)
        def _(c):
            tout[c] = plsc.load_gather(buf, [iotaT + c])

        t, s = divmod(k, CPW // TCELLS)
        cell0 = wid * CPW + s * TCELLS
        pltpu.sync_copy(tout, tbl.at[pl.ds(t * HW + cell0, TCELLS), :])

    njobs = 3 * (CPW // TCELLS)
    fire(0, *bufs[0])
    for k in range(njobs):
        if k + 1 < njobs:
            fire(k + 1, *bufs[(k + 1) % 2])
        drain(k, *bufs[k % 2])
        trans_out(k, bufs[k % 2][0])

    @pl.when(wid < 3)
    def _():
        buf, sem = bufs[0]
        for r in range(RANK):
            pltpu.async_copy(lns48.at[wid * RANK + r, :],
                             buf.at[pl.ds(r * TPITCH, GRID)], sem)
        for r in range(RANK):
            pltpu.make_async_copy(lns48.at[wid * RANK + r, :],
                                  buf.at[pl.ds(r * TPITCH, GRID)],
                                  sem).wait()

        @plsc.parallel_loop(0, GRID, unroll=8)
        def _(c):
            tout[c] = plsc.load_gather(buf, [iotaT + c])

        pltpu.sync_copy(tout.at[pl.ds(0, GRID), :],
                        tbl.at[pl.ds(3 * HW + wid * GRID, GRID), :])


_transpose = functools.partial(
    pl.kernel,
    out_type=jax.ShapeDtypeStruct((3 * HW + 3 * GRID, RANK), jnp.float32),
    mesh=plsc.VectorSubcoreMesh(
        core_axis_name="c", subcore_axis_name="s",
        num_cores=NUM_CORES, num_subcores=NUM_SUBCORES),
    scratch_types=(
        [pltpu.VMEM((RANK * TPITCH,), jnp.float32)] * 2
        + [pltpu.VMEM((TCELLS, RANK), jnp.float32)]
        + [pltpu.SemaphoreType.DMA] * 2
    ),
    compiler_params=pltpu.CompilerParams(use_tc_tiling_on_sc=False, needs_layout_passes=False),
)(_tbody)


def _body(xT, tbl, out, coords, vLL, ib0, wb0, ib1, wb1, *rest):
    ix0 = rest[0:12]
    ix1 = rest[12:24]
    gA = rest[24:36]
    gB = rest[36:48]
    oA = rest[48:51]
    oB = rest[51:54]
    sem0, sem1, semoA, semoB = rest[54:58]

    cid = lax.axis_index("c")
    sid = lax.axis_index("s")
    wid = sid * NUM_CORES + cid
    base = wid * PPW

    pltpu.sync_copy(xT.at[:, pl.ds(base, PPW)], coords)
    pltpu.sync_copy(tbl.at[pl.ds(3 * HW, 3 * GRID), :], vLL)

    slots = (
        (ib0, wb0, ix0, gA, oA, sem0, semoA),
        (ib1, wb1, ix1, gB, oB, sem1, semoB),
    )

    def s1_fire(cbase, slot):
        ibase, wbase, pidx, gbufs, _, sem, _ = slot

        def s1(i):
            off = i * LANES
            sl = pl.ds(off, LANES)

            def axis_setup(coord):
                t = coord + 1.0
                f = (t * float(GRID) - 1.0) * 0.5
                i0 = f.astype(jnp.int32)
                w1 = f - i0.astype(jnp.float32)
                w0 = 1.0 - w1
                i1 = i0 + 1
                ok = i1 < GRID
                return i0, jnp.where(ok, i1, GRID - 1), w0, jnp.where(ok, w1, 0.0)

            cx = coords[0, pl.ds(cbase + off, LANES)]
            cy = coords[1, pl.ds(cbase + off, LANES)]
            cz = coords[2, pl.ds(cbase + off, LANES)]
            x0, x1, wx0, wx1 = axis_setup(cx)
            y0, y1, wy0, wy1 = axis_setup(cy)
            z0, z1, wz0, wz1 = axis_setup(cz)

            ibase[0, sl] = x0
            ibase[1, sl] = x1
            ibase[2, sl] = y0
            ibase[3, sl] = y1
            ibase[4, sl] = z0
            ibase[5, sl] = z1
            wbase[0, sl] = wx0
            wbase[1, sl] = wx1
            wbase[2, sl] = wy0
            wbase[3, sl] = wy1
            wbase[4, sl] = wz0
            wbase[5, sl] = wz1

            yr0 = y0 * GRID
            yr1 = y1 * GRID
            zr0 = z0 * GRID + HW
            zr1 = z1 * GRID + HW
            zs0 = z0 * GRID + 2 * HW
            zs1 = z1 * GRID + 2 * HW
            pidx[0][sl] = yr0 + x0
            pidx[1][sl] = yr0 + x1
            pidx[2][sl] = yr1 + x0
            pidx[3][sl] = yr1 + x1
            pidx[4][sl] = zr0 + x0
            pidx[5][sl] = zr0 + x1
            pidx[6][sl] = zr1 + x0
            pidx[7][sl] = zr1 + x1
            pidx[8][sl] = zs0 + y0
            pidx[9][sl] = zs0 + y1
            pidx[10][sl] = zs1 + y0
            pidx[11][sl] = zs1 + y1

        plsc.parallel_loop(0, CHUNK // LANES, unroll=2)(s1)
        for ix, gb in zip(pidx, gbufs):
            pltpu.async_copy(tbl.at[ix], gb, sem)

    def drain(slot):
        _, _, pidx, gbufs, _, sem, _ = slot
        for ix, gb in zip(pidx, gbufs):
            pltpu.make_async_copy(tbl.at[ix], gb, sem).wait()

    def s2_out(cbase, slot):
        ibase, wbase, _, gbufs, obufs, _, semo = slot
        o0, o1, o2 = obufs
        gout = base + cbase

        @pl.when(cbase >= 2 * CHUNK)
        def _():
            pltpu.make_async_copy(o0, out.at[0, pl.ds(gout, CHUNK)], semo).wait()
            pltpu.make_async_copy(o1, out.at[1, pl.ds(gout, CHUNK)], semo).wait()
            pltpu.make_async_copy(o2, out.at[2, pl.ds(gout, CHUNK)], semo).wait()
        gb0, gb1, gb2, gb3, gb4, gb5, gb6, gb7, gb8, gb9, gb10, gb11 = gbufs

        def s2(g):
            sl = pl.ds(g * LANES, LANES)
            x0v = ibase[0, sl]
            x1v = ibase[1, sl]
            y0v = ibase[2, sl]
            y1v = ibase[3, sl]
            z0v = ibase[4, sl]
            z1v = ibase[5, sl]
            wx0v = wbase[0, sl]
            wx1v = wbase[1, sl]
            wy0v = wbase[2, sl]
            wy1v = wbase[3, sl]
            wz0v = wbase[4, sl]
            wz1v = wbase[5, sl]

            for j in range(LANES):
                p = g * LANES + j
                wx0 = wx0v[j]
                wx1 = wx1v[j]
                wy0 = wy0v[j]
                wy1 = wy1v[j]
                wz0 = wz0v[j]
                wz1 = wz1v[j]

                xy = ((wx0 * wy0) * gb0[p] + (wx1 * wy0) * gb1[p]
                      + (wx0 * wy1) * gb2[p] + (wx1 * wy1) * gb3[p])
                fz = wz0 * vLL[z0v[j]] + wz1 * vLL[z1v[j]]
                o0[p] = xy * fz

                xz = ((wx0 * wz0) * gb4[p] + (wx1 * wz0) * gb5[p]
                      + (wx0 * wz1) * gb6[p] + (wx1 * wz1) * gb7[p])
                fy = wy0 * vLL[GRID + y0v[j]] + wy1 * vLL[GRID + y1v[j]]
                o1[p] = xz * fy

                yz = ((wy0 * wz0) * gb8[p] + (wy1 * wz0) * gb9[p]
                      + (wy0 * wz1) * gb10[p] + (wy1 * wz1) * gb11[p])
                fx = (wx0 * vLL[2 * GRID + x0v[j]]
                      + wx1 * vLL[2 * GRID + x1v[j]])
                o2[p] = yz * fx

        plsc.parallel_loop(0, CHUNK // LANES, unroll=2)(s2)

        pltpu.async_copy(o0, out.at[0, pl.ds(gout, CHUNK)], semo)
        pltpu.async_copy(o1, out.at[1, pl.ds(gout, CHUNK)], semo)
        pltpu.async_copy(o2, out.at[2, pl.ds(gout, CHUNK)], semo)

    s1_fire(0, slots[0])

    def step(it, carry):
        ch0 = it * 2 * CHUNK
        ch1 = ch0 + CHUNK
        s1_fire(ch1, slots[1])
        drain(slots[0])
        s2_out(ch0, slots[0])

        @pl.when(it < NCHUNK // 2 - 1)
        def _():
            s1_fire(ch1 + CHUNK, slots[0])

        drain(slots[1])
        s2_out(ch1, slots[1])
        return carry

    lax.fori_loop(0, NCHUNK // 2, step, 0)

    for sl_ in slots:
        _, _, _, _, (p0, p1, p2), _, semo = sl_
        pltpu.make_async_copy(p0, out.at[0, pl.ds(base, CHUNK)], semo).wait()
        pltpu.make_async_copy(p1, out.at[1, pl.ds(base, CHUNK)], semo).wait()
        pltpu.make_async_copy(p2, out.at[2, pl.ds(base, CHUNK)], semo).wait()


_grid_sampler = functools.partial(
    pl.kernel,
    out_type=jax.ShapeDtypeStruct((3, N_PTS, RANK), jnp.float32),
    mesh=plsc.VectorSubcoreMesh(
        core_axis_name="c", subcore_axis_name="s",
        num_cores=NUM_CORES, num_subcores=NUM_SUBCORES),
    scratch_types=(
        [pltpu.VMEM((3, PPW), jnp.float32)]
        + [pltpu.VMEM((3 * GRID, RANK), jnp.float32)]
        + [pltpu.VMEM((6, CHUNK), jnp.int32),
           pltpu.VMEM((6, CHUNK), jnp.float32),
           pltpu.VMEM((6, CHUNK), jnp.int32),
           pltpu.VMEM((6, CHUNK), jnp.float32)]
        + [pltpu.VMEM((CHUNK,), jnp.int32)] * 24
        + [pltpu.VMEM((CHUNK, RANK), jnp.float32)] * 24
        + [pltpu.VMEM((CHUNK, RANK), jnp.float32)] * 6
        + [pltpu.SemaphoreType.DMA] * 4
    ),
    compiler_params=pltpu.CompilerParams(use_tc_tiling_on_sc=False, needs_layout_passes=False),
)(_body)


def kernel(x, feats_XY, feats_Z, feats_XZ, feats_Y, feats_YZ, feats_X):
    xT = x.T
    pXY = feats_XY.reshape(RANK, HW)
    pXZ = feats_XZ.reshape(RANK, HW)
    pYZ = feats_YZ.reshape(RANK, HW)
    lns48 = jnp.concatenate(
        [feats_Z[:, :, 0], feats_Y[:, :, 0], feats_X[:, :, 0]], axis=0)
    tbl = _transpose(pXY, pXZ, pYZ, lns48)
    return _grid_sampler(xT, tbl)


# speedup vs baseline: 1.0091x; 1.0091x over previous
import functools

import jax
import jax.numpy as jnp
from jax import lax
from jax.experimental import pallas as pl
from jax.experimental.pallas import tpu as pltpu
from jax.experimental.pallas import tpu_sc as plsc

RANK = 16
GRID = 512
N_PTS = 262144
HW = GRID * GRID
NUM_CORES = 2
NUM_SUBCORES = 16
NUM_WORKERS = NUM_CORES * NUM_SUBCORES
PPW = N_PTS // NUM_WORKERS
CHUNK = 128
NCHUNK = PPW // CHUNK
LANES = 16
TCELLS = 2048
TPITCH = TCELLS + 8
CPW = HW // NUM_WORKERS


def _tbody(pXY, pXZ, pYZ, lns48, tbl, tinA, tinB, tout, semA, semB):
    cid = lax.axis_index("c")
    sid = lax.axis_index("s")
    wid = sid * NUM_CORES + cid
    planes = (pXY, pXZ, pYZ)
    bufs = ((tinA, semA), (tinB, semB))
    iotaT = lax.iota(jnp.int32, LANES) * TPITCH

    def fire(k, buf, sem):
        t, s = divmod(k, CPW // TCELLS)
        pref = planes[t]
        cell0 = wid * CPW + s * TCELLS
        for r in range(RANK):
            pltpu.async_copy(pref.at[r, pl.ds(cell0, TCELLS)],
                             buf.at[pl.ds(r * TPITCH, TCELLS)], sem)

    def drain(k, buf, sem):
        t, s = divmod(k, CPW // TCELLS)
        pref = planes[t]
        cell0 = wid * CPW + s * TCELLS
        for r in range(RANK):
            pltpu.make_async_copy(pref.at[r, pl.ds(cell0, TCELLS)],
                                  buf.at[pl.ds(r * TPITCH, TCELLS)],
                                  sem).wait()

    def trans_out(k, buf):
        @plsc.parallel_loop(0, TCELLS, unroll=8)
        def _(c):
            tout[c] = plsc.load_gather(buf, [iotaT + c])

        t, s = divmod(k, CPW // TCELLS)
        cell0 = wid * CPW + s * TCELLS
        pltpu.sync_copy(tout, tbl.at[pl.ds(t * HW + cell0, TCELLS), :])

    njobs = 3 * (CPW // TCELLS)
    fire(0, *bufs[0])
    for k in range(njobs):
        if k + 1 < njobs:
            fire(k + 1, *bufs[(k + 1) % 2])
        drain(k, *bufs[k % 2])
        trans_out(k, bufs[k % 2][0])

    @pl.when(wid < 3)
    def _():
        buf, sem = bufs[0]
        for r in range(RANK):
            pltpu.async_copy(lns48.at[wid * RANK + r, :],
                             buf.at[pl.ds(r * TPITCH, GRID)], sem)
        for r in range(RANK):
            pltpu.make_async_copy(lns48.at[wid * RANK + r, :],
                                  buf.at[pl.ds(r * TPITCH, GRID)],
                                  sem).wait()

        @plsc.parallel_loop(0, GRID, unroll=8)
        def _(c):
            tout[c] = plsc.load_gather(buf, [iotaT + c])

        pltpu.sync_copy(tout.at[pl.ds(0, GRID), :],
                        tbl.at[pl.ds(3 * HW + wid * GRID, GRID), :])


_transpose = functools.partial(
    pl.kernel,
    out_type=jax.ShapeDtypeStruct((3 * HW + 3 * GRID, RANK), jnp.float32),
    mesh=plsc.VectorSubcoreMesh(
        core_axis_name="c", subcore_axis_name="s",
        num_cores=NUM_CORES, num_subcores=NUM_SUBCORES),
    scratch_types=(
        [pltpu.VMEM((RANK * TPITCH,), jnp.float32)] * 2
        + [pltpu.VMEM((TCELLS, RANK), jnp.float32)]
        + [pltpu.SemaphoreType.DMA] * 2
    ),
    compiler_params=pltpu.CompilerParams(use_tc_tiling_on_sc=False, needs_layout_passes=False),
)(_tbody)


def _body(xT, tbl, out, coords, vLL, ib0, wb0, ib1, wb1, *rest):
    ix0 = rest[0:12]
    ix1 = rest[12:24]
    gA = rest[24:36]
    gB = rest[36:48]
    oA = rest[48:51]
    oB = rest[51:54]
    sem0, sem1 = rest[54:56]

    cid = lax.axis_index("c")
    sid = lax.axis_index("s")
    wid = sid * NUM_CORES + cid
    base = wid * PPW

    pltpu.sync_copy(xT.at[:, pl.ds(base, PPW)], coords)
    pltpu.sync_copy(tbl.at[pl.ds(3 * HW, 3 * GRID), :], vLL)

    slots = (
        (ib0, wb0, ix0, gA, oA, sem0),
        (ib1, wb1, ix1, gB, oB, sem1),
    )

    def s1_fire(cbase, slot):
        ibase, wbase, pidx, gbufs, _, sem = slot

        def s1(i):
            off = i * LANES
            sl = pl.ds(off, LANES)

            def axis_setup(coord):
                t = coord + 1.0
                f = (t * float(GRID) - 1.0) * 0.5
                i0 = f.astype(jnp.int32)
                w1 = f - i0.astype(jnp.float32)
                w0 = 1.0 - w1
                i1 = i0 + 1
                ok = i1 < GRID
                return i0, jnp.where(ok, i1, GRID - 1), w0, jnp.where(ok, w1, 0.0)

            cx = coords[0, pl.ds(cbase + off, LANES)]
            cy = coords[1, pl.ds(cbase + off, LANES)]
            cz = coords[2, pl.ds(cbase + off, LANES)]
            x0, x1, wx0, wx1 = axis_setup(cx)
            y0, y1, wy0, wy1 = axis_setup(cy)
            z0, z1, wz0, wz1 = axis_setup(cz)

            ibase[0, sl] = x0
            ibase[1, sl] = x1
            ibase[2, sl] = y0
            ibase[3, sl] = y1
            ibase[4, sl] = z0
            ibase[5, sl] = z1
            wbase[0, sl] = wx0
            wbase[1, sl] = wx1
            wbase[2, sl] = wy0
            wbase[3, sl] = wy1
            wbase[4, sl] = wz0
            wbase[5, sl] = wz1

            yr0 = y0 * GRID
            yr1 = y1 * GRID
            zr0 = z0 * GRID + HW
            zr1 = z1 * GRID + HW
            zs0 = z0 * GRID + 2 * HW
            zs1 = z1 * GRID + 2 * HW
            pidx[0][sl] = yr0 + x0
            pidx[1][sl] = yr0 + x1
            pidx[2][sl] = yr1 + x0
            pidx[3][sl] = yr1 + x1
            pidx[4][sl] = zr0 + x0
            pidx[5][sl] = zr0 + x1
            pidx[6][sl] = zr1 + x0
            pidx[7][sl] = zr1 + x1
            pidx[8][sl] = zs0 + y0
            pidx[9][sl] = zs0 + y1
            pidx[10][sl] = zs1 + y0
            pidx[11][sl] = zs1 + y1

        plsc.parallel_loop(0, CHUNK // LANES, unroll=2)(s1)
        for ix, gb in zip(pidx, gbufs):
            pltpu.async_copy(tbl.at[ix], gb, sem)

    def drain(slot):
        _, _, pidx, gbufs, _, sem = slot
        for ix, gb in zip(pidx, gbufs):
            pltpu.make_async_copy(tbl.at[ix], gb, sem).wait()

    def s2_out(cbase, slot):
        ibase, wbase, _, gbufs, obufs, _ = slot
        gb0, gb1, gb2, gb3, gb4, gb5, gb6, gb7, gb8, gb9, gb10, gb11 = gbufs
        o0, o1, o2 = obufs

        def s2(g):
            sl = pl.ds(g * LANES, LANES)
            x0v = ibase[0, sl]
            x1v = ibase[1, sl]
            y0v = ibase[2, sl]
            y1v = ibase[3, sl]
            z0v = ibase[4, sl]
            z1v = ibase[5, sl]
            wx0v = wbase[0, sl]
            wx1v = wbase[1, sl]
            wy0v = wbase[2, sl]
            wy1v = wbase[3, sl]
            wz0v = wbase[4, sl]
            wz1v = wbase[5, sl]

            for j in range(LANES):
                p = g * LANES + j
                wx0 = wx0v[j]
                wx1 = wx1v[j]
                wy0 = wy0v[j]
                wy1 = wy1v[j]
                wz0 = wz0v[j]
                wz1 = wz1v[j]

                xy = ((wx0 * wy0) * gb0[p] + (wx1 * wy0) * gb1[p]
                      + (wx0 * wy1) * gb2[p] + (wx1 * wy1) * gb3[p])
                fz = wz0 * vLL[z0v[j]] + wz1 * vLL[z1v[j]]
                o0[p] = xy * fz

                xz = ((wx0 * wz0) * gb4[p] + (wx1 * wz0) * gb5[p]
                      + (wx0 * wz1) * gb6[p] + (wx1 * wz1) * gb7[p])
                fy = wy0 * vLL[GRID + y0v[j]] + wy1 * vLL[GRID + y1v[j]]
                o1[p] = xz * fy

                yz = ((wy0 * wz0) * gb8[p] + (wy1 * wz0) * gb9[p]
                      + (wy0 * wz1) * gb10[p] + (wy1 * wz1) * gb11[p])
                fx = (wx0 * vLL[2 * GRID + x0v[j]]
                      + wx1 * vLL[2 * GRID + x1v[j]])
                o2[p] = yz * fx

        plsc.parallel_loop(0, CHUNK // LANES)(s2)

        gout = base + cbase
        pltpu.sync_copy(o0, out.at[0, pl.ds(gout, CHUNK)])
        pltpu.sync_copy(o1, out.at[1, pl.ds(gout, CHUNK)])
        pltpu.sync_copy(o2, out.at[2, pl.ds(gout, CHUNK)])

    s1_fire(0, slots[0])

    def step(it, carry):
        ch0 = it * 2 * CHUNK
        ch1 = ch0 + CHUNK
        s1_fire(ch1, slots[1])
        drain(slots[0])
        s2_out(ch0, slots[0])

        @pl.when(it < NCHUNK // 2 - 1)
        def _():
            s1_fire(ch1 + CHUNK, slots[0])

        drain(slots[1])
        s2_out(ch1, slots[1])
        return carry

    lax.fori_loop(0, NCHUNK // 2, step, 0)


_grid_sampler = functools.partial(
    pl.kernel,
    out_type=jax.ShapeDtypeStruct((3, N_PTS, RANK), jnp.float32),
    mesh=plsc.VectorSubcoreMesh(
        core_axis_name="c", subcore_axis_name="s",
        num_cores=NUM_CORES, num_subcores=NUM_SUBCORES),
    scratch_types=(
        [pltpu.VMEM((3, PPW), jnp.float32)]
        + [pltpu.VMEM((3 * GRID, RANK), jnp.float32)]
        + [pltpu.VMEM((6, CHUNK), jnp.int32),
           pltpu.VMEM((6, CHUNK), jnp.float32),
           pltpu.VMEM((6, CHUNK), jnp.int32),
           pltpu.VMEM((6, CHUNK), jnp.float32)]
        + [pltpu.VMEM((CHUNK,), jnp.int32)] * 24
        + [pltpu.VMEM((CHUNK, RANK), jnp.float32)] * 24
        + [pltpu.VMEM((CHUNK, RANK), jnp.float32)] * 6
        + [pltpu.SemaphoreType.DMA] * 2
    ),
    compiler_params=pltpu.CompilerParams(use_tc_tiling_on_sc=False, needs_layout_passes=False),
)(_body)


def kernel(x, feats_XY, feats_Z, feats_XZ, feats_Y, feats_YZ, feats_X):
    xT = x.T
    pXY = feats_XY.reshape(RANK, HW)
    pXZ = feats_XZ.reshape(RANK, HW)
    pYZ = feats_YZ.reshape(RANK, HW)
    lns48 = jnp.concatenate(
        [feats_Z[:, :, 0], feats_Y[:, :, 0], feats_X[:, :, 0]], axis=0)
    tbl = _transpose(pXY, pXZ, pYZ, lns48)
    return _grid_sampler(xT, tbl)
